# Initial kernel scaffold; baseline (speedup 1.0000x reference)
import jax
import jax.numpy as jnp
from jax.experimental import pallas as pl


def kernel(x, edge_index, Wt1a, bt1a, Wg1, bg1, Wt1b, bt1b, Wt2a, bt2a, Wg2, bg2, Wt2b, bt2b, W1, b1, W2, b2):
    raise NotImplementedError("write your pallas kernel here")


# speedup vs baseline: 12.7312x; 12.7312x over previous
import functools

import jax
import jax.numpy as jnp
from jax import lax
from jax.experimental import pallas as pl
from jax.experimental.pallas import tpu as pltpu
from jax.experimental.pallas import tpu_sc as plsc

N_NODES = 10000
N_PAD = 10240
E_TOTAL = 160000
T0 = 24

F_CHUNK = 128
N_CHUNKS = 3
F_PAD = F_CHUNK * N_CHUNKS

_NSUB = 16
_ROWS_PER_SUB = N_PAD // _NSUB

_SP_KB = 100
_SP_IT = E_TOTAL // (2 * _NSUB) // _SP_KB

_DG_E = 5008
_DG_IT = _DG_E // 16
_DG_PAD = 2 * _NSUB * _DG_E - E_TOTAL


def _tconv_big(W, b, t_in):
    ks, ci, co = W.shape
    t_out = t_in - ks + 1
    big = jnp.zeros((t_in * ci, t_out * co), jnp.float32)
    for t in range(t_out):
        for k in range(ks):
            big = big.at[(t + k) * ci:(t + k + 1) * ci, t * co:(t + 1) * co].add(W[k])
    bigb = jnp.tile(b, (t_out,))[None, :]
    return big, bigb


def _gcn_big(W, b, t):
    big = jnp.kron(jnp.eye(t, dtype=jnp.float32), W)
    bigb = jnp.tile(b, (t,))[None, :]
    return big, bigb


def _pad_to(a, rows, cols):
    return jnp.pad(a, ((0, rows - a.shape[0]), (0, cols - a.shape[1])))


def _deg_call(dst2, zero_hbm):
    mesh = plsc.VectorSubcoreMesh(core_axis_name="c", subcore_axis_name="s",
                                  num_cores=2, num_subcores=_NSUB)

    @functools.partial(
        pl.kernel,
        out_type=(jax.ShapeDtypeStruct((N_PAD,), jnp.float32),
                  jax.ShapeDtypeStruct((N_PAD,), jnp.float32)),
        mesh=mesh,
        compiler_params=pltpu.CompilerParams(needs_layout_passes=False),
        scratch_types=[
            pltpu.VMEM((_DG_E,), jnp.int32),
            pltpu.VMEM((1, N_PAD), jnp.float32),
            pltpu.VMEM((_NSUB, 1, _ROWS_PER_SUB), jnp.float32),
            pltpu.VMEM((_ROWS_PER_SUB,), jnp.float32),
            pltpu.VMEM_SHARED((_NSUB, 1, N_PAD), jnp.float32),
        ],
    )
    def deg_kernel(dst_hbm, zero_h, outa, outb,
                   idx_v, hist_v, stripe_v, red_v, red_sh):
        c = lax.axis_index("c")
        s = lax.axis_index("s")
        w = c * _NSUB + s
        row0 = s * _ROWS_PER_SUB
        pltpu.sync_copy(zero_h, hist_v)
        pltpu.sync_copy(dst_hbm.at[w], idx_v)
        ones = jnp.ones((16,), jnp.float32)

        zero16 = jnp.zeros((16,), jnp.int32)

        def it(i, carry):
            idx = idx_v[pl.ds(i * 16, 16)]
            plsc.addupdate_scatter(hist_v, [zero16, idx], ones)
            return carry

        lax.fori_loop(0, _DG_IT, it, 0)
        pltpu.sync_copy(hist_v, red_sh.at[s])
        plsc.subcore_barrier()
        pltpu.sync_copy(red_sh.at[:, :, pl.ds(row0, _ROWS_PER_SUB)], stripe_v)

        def red(j, carry):
            col = pl.ds(j * 16, 16)
            acc = stripe_v[0, 0, col]
            for r in range(1, _NSUB):
                acc = acc + stripe_v[r, 0, col]
            red_v[col] = acc
            return carry

        lax.fori_loop(0, _ROWS_PER_SUB // 16, red, 0)

        @pl.when(c == 0)
        def _():
            pltpu.sync_copy(red_v, outa.at[pl.ds(row0, _ROWS_PER_SUB)])

        @pl.when(c == 1)
        def _():
            pltpu.sync_copy(red_v, outb.at[pl.ds(row0, _ROWS_PER_SUB)])

    return deg_kernel(dst2, zero_hbm)


def _spmm_call(h0, h1, h2, src3, dst3, zero_hbm):
    mesh = plsc.VectorSubcoreMesh(core_axis_name="c", subcore_axis_name="s",
                                  num_cores=2, num_subcores=_NSUB)

    @functools.partial(
        pl.kernel,
        out_type=tuple(jax.ShapeDtypeStruct((N_PAD, F_CHUNK), jnp.float32)
                       for _ in range(2 * N_CHUNKS)),
        mesh=mesh,
        scratch_types=[
            pltpu.VMEM((_SP_IT, _SP_KB), jnp.int32),
            pltpu.VMEM((_SP_IT, _SP_KB), jnp.int32),
            pltpu.VMEM((_SP_KB, F_CHUNK), jnp.float32),
            pltpu.VMEM_SHARED((N_PAD, F_CHUNK), jnp.float32),
            pltpu.SemaphoreType.DMA,
        ],
    )
    def spmm_kernel(ha, hb, hc, src_h, dst_h, zero_h,
                    oa0, oa1, oa2, ob0, ob1, ob2,
                    src_v, dst_v, rows_v, agg_sh, sem):
        c = lax.axis_index("c")
        s = lax.axis_index("s")
        w = c * _NSUB + s
        row0 = s * _ROWS_PER_SUB
        stripe = pl.ds(row0, _ROWS_PER_SUB)
        pltpu.sync_copy(src_h.at[w], src_v)
        pltpu.sync_copy(dst_h.at[w], dst_v)
        pltpu.sync_copy(zero_h, agg_sh.at[stripe])
        plsc.subcore_barrier()

        def run_chunk(h_hbm, out_a, out_b, last):
            def it(i, carry):
                pltpu.async_copy(h_hbm.at[src_v.at[i]], rows_v, sem).wait()
                pltpu.sync_copy(rows_v, agg_sh.at[dst_v.at[i]], add=True)
                return carry

            lax.fori_loop(0, _SP_IT, it, 0)
            plsc.subcore_barrier()

            @pl.when(c == 0)
            def _():
                pltpu.sync_copy(agg_sh.at[stripe], out_a.at[stripe])

            @pl.when(c == 1)
            def _():
                pltpu.sync_copy(agg_sh.at[stripe], out_b.at[stripe])

            if not last:
                pltpu.sync_copy(zero_h, agg_sh.at[stripe])
                plsc.subcore_barrier()

        run_chunk(ha, oa0, ob0, False)
        run_chunk(hb, oa1, ob1, False)
        run_chunk(hc, oa2, ob2, True)

    return spmm_kernel(h0, h1, h2, src3, dst3, zero_hbm)


_NB = 512
_MB = 40


def _norm_of(dega, degb):
    deg = dega[...] + degb[...]
    return lax.rsqrt(jnp.maximum(deg, 1.0))


def _mm1_body(x_ref, dega_ref, degb_ref, w_ref, b_ref,
              out0_ref, out1_ref, out2_ref):
    h = jnp.dot(x_ref[...], w_ref[...], preferred_element_type=jnp.float32)
    h = jnp.maximum(h + b_ref[...], 0.0)
    h = h * _norm_of(dega_ref[...], degb_ref[...])
    out0_ref[...] = h[:, :F_CHUNK]
    out1_ref[...] = h[:, F_CHUNK:2 * F_CHUNK]
    out2_ref[...] = h[:, 2 * F_CHUNK:]


def _mm1_call(x2d, dega, degb, w_big, b_big):
    grid = (N_PAD // _NB,)
    return pl.pallas_call(
        _mm1_body,
        grid=grid,
        in_specs=[
            pl.BlockSpec((_NB, T0), lambda i: (i, 0)),
            pl.BlockSpec((_NB, 1), lambda i: (i, 0)),
            pl.BlockSpec((_NB, 1), lambda i: (i, 0)),
            pl.BlockSpec((T0, F_PAD), lambda i: (0, 0)),
            pl.BlockSpec((1, F_PAD), lambda i: (0, 0)),
        ],
        out_specs=[
            pl.BlockSpec((_NB, F_CHUNK), lambda i: (i, 0)),
            pl.BlockSpec((_NB, F_CHUNK), lambda i: (i, 0)),
            pl.BlockSpec((_NB, F_CHUNK), lambda i: (i, 0)),
        ],
        out_shape=[jax.ShapeDtypeStruct((N_PAD, F_CHUNK), jnp.float32)] * 3,
    )(x2d, dega, degb, w_big, b_big)


def _combine(parts, norm):
    a0 = parts[0][...] + parts[3][...]
    a1 = parts[1][...] + parts[4][...]
    a2 = parts[2][...] + parts[5][...]
    return jnp.concatenate([a0, a1, a2], axis=1) * norm


def _mm2_body(p0, p1, p2, p3, p4, p5, dega_ref, degb_ref,
              wg_ref, bg_ref, wb_ref, bb_ref, wa2_ref, ba2_ref,
              out0_ref, out1_ref, out2_ref):
    norm = _norm_of(dega_ref[...], degb_ref[...])
    a = _combine((p0, p1, p2, p3, p4, p5), norm)
    g = jnp.maximum(jnp.dot(a, wg_ref[...], preferred_element_type=jnp.float32)
                    + bg_ref[...], 0.0)
    h = jnp.dot(g, wb_ref[...], preferred_element_type=jnp.float32) + bb_ref[...]
    h = jnp.where(h > 0.0, h, jnp.exp(jnp.minimum(h, 0.0)) - 1.0)
    t = jnp.maximum(jnp.dot(h, wa2_ref[...], preferred_element_type=jnp.float32)
                    + ba2_ref[...], 0.0)
    t = t * norm
    out0_ref[...] = t[:, :F_CHUNK]
    out1_ref[...] = t[:, F_CHUNK:2 * F_CHUNK]
    out2_ref[...] = t[:, 2 * F_CHUNK:]


def _mm2_call(parts, dega, degb, wg, bg, wb, bb, wa2, ba2):
    f_mid = wb.shape[1]
    grid = (N_PAD // _NB,)
    part_spec = pl.BlockSpec((_NB, F_CHUNK), lambda i: (i, 0))
    return pl.pallas_call(
        _mm2_body,
        grid=grid,
        in_specs=[part_spec] * 6 + [
            pl.BlockSpec((_NB, 1), lambda i: (i, 0)),
            pl.BlockSpec((_NB, 1), lambda i: (i, 0)),
            pl.BlockSpec((F_PAD, F_PAD), lambda i: (0, 0)),
            pl.BlockSpec((1, F_PAD), lambda i: (0, 0)),
            pl.BlockSpec((F_PAD, f_mid), lambda i: (0, 0)),
            pl.BlockSpec((1, f_mid), lambda i: (0, 0)),
            pl.BlockSpec((f_mid, F_PAD), lambda i: (0, 0)),
            pl.BlockSpec((1, F_PAD), lambda i: (0, 0)),
        ],
        out_specs=[
            pl.BlockSpec((_NB, F_CHUNK), lambda i: (i, 0)),
            pl.BlockSpec((_NB, F_CHUNK), lambda i: (i, 0)),
            pl.BlockSpec((_NB, F_CHUNK), lambda i: (i, 0)),
        ],
        out_shape=[jax.ShapeDtypeStruct((N_PAD, F_CHUNK), jnp.float32)] * 3,
    )(*parts, dega, degb, wg, bg, wb, bb, wa2, ba2)


def _mlp_body(p0, p1, p2, p3, p4, p5, dega_ref, degb_ref,
              wg_ref, bg_ref, wb_ref, bb_ref,
              w1_ref, b1_ref, w2_ref, b2_ref, out_ref):
    norm = _norm_of(dega_ref[...], degb_ref[...])
    a = _combine((p0, p1, p2, p3, p4, p5), norm)
    g = jnp.maximum(jnp.dot(a, wg_ref[...], preferred_element_type=jnp.float32)
                    + bg_ref[...], 0.0)
    feat = jnp.dot(g, wb_ref[...], preferred_element_type=jnp.float32) + bb_ref[...]
    hid = jnp.sum(feat[:, :, None] * w1_ref[...], axis=1) + b1_ref[...]
    hid = jnp.maximum(hid, 0.0)
    out = jnp.sum(hid[:, :, None] * w2_ref[...], axis=1) + b2_ref[...]
    out_ref[...] = out


def _mlp_call(parts, dega, degb, wg, bg, wb, bb, w1, b1, w2, b2):
    f_feat = wb.shape[1]
    f_hid = w1.shape[2]
    f_out = w2.shape[2]
    grid = (N_NODES // _MB,)
    part_spec = pl.BlockSpec((_MB, F_CHUNK), lambda i: (i, 0))
    return pl.pallas_call(
        _mlp_body,
        grid=grid,
        in_specs=[part_spec] * 6 + [
            pl.BlockSpec((_MB, 1), lambda i: (i, 0)),
            pl.BlockSpec((_MB, 1), lambda i: (i, 0)),
            pl.BlockSpec((F_PAD, F_PAD), lambda i: (0, 0)),
            pl.BlockSpec((1, F_PAD), lambda i: (0, 0)),
            pl.BlockSpec((F_PAD, f_feat), lambda i: (0, 0)),
            pl.BlockSpec((1, f_feat), lambda i: (0, 0)),
            pl.BlockSpec((_MB, f_feat, f_hid), lambda i: (i, 0, 0)),
            pl.BlockSpec((_MB, f_hid), lambda i: (i, 0)),
            pl.BlockSpec((_MB, f_hid, f_out), lambda i: (i, 0, 0)),
            pl.BlockSpec((_MB, f_out), lambda i: (i, 0)),
        ],
        out_specs=pl.BlockSpec((_MB, f_out), lambda i: (i, 0)),
        out_shape=jax.ShapeDtypeStruct((N_NODES, f_out), jnp.float32),
    )(*parts, dega, degb, wg, bg, wb, bb, w1, b1, w2, b2)


def kernel(x, edge_index, Wt1a, bt1a, Wg1, bg1, Wt1b, bt1b,
           Wt2a, bt2a, Wg2, bg2, Wt2b, bt2b, W1, b1, W2, b2):
    w1a_big, b1a_big = _tconv_big(Wt1a, bt1a, 24)
    wg1_big, bg1_big = _gcn_big(Wg1, bg1, 22)
    w1b_big, b1b_big = _tconv_big(Wt1b, bt1b, 22)
    w2a_big, b2a_big = _tconv_big(Wt2a, bt2a, 20)
    wg2_big, bg2_big = _gcn_big(Wg2, bg2, 18)
    w2b_big, b2b_big = _tconv_big(Wt2b, bt2b, 18)

    w1a_big = _pad_to(w1a_big, T0, F_PAD)
    b1a_big = _pad_to(b1a_big, 1, F_PAD)
    wg1_big = _pad_to(wg1_big, F_PAD, F_PAD)
    bg1_big = _pad_to(bg1_big, 1, F_PAD)
    w1b_big = _pad_to(w1b_big, F_PAD, 640)
    w2a_big = _pad_to(w2a_big, 640, F_PAD)
    b2a_big = _pad_to(b2a_big, 1, F_PAD)
    wg2_big = _pad_to(wg2_big, F_PAD, F_PAD)
    bg2_big = _pad_to(bg2_big, 1, F_PAD)
    w2b_big = _pad_to(w2b_big, F_PAD, 512)

    x2d = jnp.pad(jnp.transpose(x[0]), ((0, N_PAD - N_NODES), (0, 0)))
    src = edge_index[0]
    dst = edge_index[1]
    dst2_dg = jnp.concatenate(
        [dst, jnp.full((_DG_PAD,), N_PAD - 1, jnp.int32)]).reshape(
            2 * _NSUB, _DG_E)
    src3_sp = src.reshape(2 * _NSUB, _SP_IT, _SP_KB)
    dst3_sp = dst.reshape(2 * _NSUB, _SP_IT, _SP_KB)

    zero_dg = jnp.zeros((1, N_PAD), jnp.float32)
    zero_sp = jnp.zeros((_ROWS_PER_SUB, F_CHUNK), jnp.float32)

    dega, degb = _deg_call(dst2_dg, zero_dg)
    dega = dega[:, None]
    degb = degb[:, None]

    hp = _mm1_call(x2d, dega, degb, w1a_big, b1a_big)

    agg1 = _spmm_call(hp[0], hp[1], hp[2], src3_sp, dst3_sp, zero_sp)

    hq = _mm2_call(agg1, dega, degb,
                   wg1_big, bg1_big, w1b_big, b1b_big, w2a_big, b2a_big)

    agg2 = _spmm_call(hq[0], hq[1], hq[2], src3_sp, dst3_sp, zero_sp)

    out = _mlp_call(agg2, dega, degb,
                    wg2_big, bg2_big, w2b_big, b2b_big, W1, b1, W2, b2)

    return out[:, None, :]
